# Initial kernel scaffold; baseline (speedup 1.0000x reference)
import functools

import jax
import jax.numpy as jnp
from jax import lax
from jax.experimental import pallas as pl
from jax.experimental.pallas import tpu as pltpu
from jax.experimental.pallas import tpu_sc as plsc

NC = 2
NS = 16
NW = NC * NS
B = 128
BLK = 128

_MESH = functools.partial(
    plsc.VectorSubcoreMesh,
    core_axis_name="c", subcore_axis_name="s", num_cores=NC, num_subcores=NS,
)


def _worker_id():
    return lax.axis_index("s") * NC + lax.axis_index("c")


def _make_pass_a(N, D, NP, EPT):
    RPS = NP // NS
    NBX = NP // B
    TX = (NBX + NW - 1) // NW
    NBE = EPT // B

    @functools.partial(
        pl.kernel,
        out_type=(
            jax.ShapeDtypeStruct((NP, D), jnp.float32),
            jax.ShapeDtypeStruct((NC, NP, 8), jnp.float32),
        ),
        mesh=_MESH(),
        scratch_types=[
            pltpu.VMEM((B,), jnp.int32),
            pltpu.VMEM((B, D), jnp.float32),
            pltpu.VMEM((B, 8), jnp.float32),
            pltpu.VMEM_SHARED((NP, 8), jnp.float32),
            pltpu.SemaphoreType.DMA,
        ],
    )
    def pass_a(emb_h, xind_h, col_h, ones_h, zer8_h, x_h, degp_h,
               idxv, rows, ones_v, deg_acc, sem):
        c = lax.axis_index("c")
        s = lax.axis_index("s")
        w = _worker_id()
        pltpu.sync_copy(zer8_h, deg_acc.at[pl.ds(s * RPS, RPS)])
        pltpu.sync_copy(ones_h, ones_v)
        plsc.subcore_barrier()

        def xgather(t, carry):
            bid = w + NW * t

            @pl.when(bid < NBX)
            def _():
                pltpu.sync_copy(xind_h.at[pl.ds(bid * B, B)], idxv)
                pltpu.async_copy(emb_h.at[idxv], rows, sem).wait()
                pltpu.sync_copy(rows, x_h.at[pl.ds(bid * B, B)])
            return carry

        lax.fori_loop(0, TX, xgather, 0)

        def deg_scatter(bi, carry):
            base = w * EPT + bi * B
            pltpu.sync_copy(col_h.at[pl.ds(base, B)], idxv)
            pltpu.sync_copy(ones_v, deg_acc.at[idxv], add=True)
            return carry

        lax.fori_loop(0, NBE, deg_scatter, 0)
        plsc.subcore_barrier()
        pltpu.sync_copy(deg_acc.at[pl.ds(s * RPS, RPS)],
                        degp_h.at[c, pl.ds(s * RPS, RPS)])

    return pass_a


def _make_prop(D, NP, EPT):
    RPS = NP // NS
    NBE = EPT // B

    @functools.partial(
        pl.kernel,
        out_type=jax.ShapeDtypeStruct((NC, NP, D), jnp.float32),
        mesh=_MESH(),
        scratch_types=[
            pltpu.VMEM((B,), jnp.int32),
            pltpu.VMEM((B,), jnp.int32),
            pltpu.VMEM((B, D), jnp.float32),
            pltpu.VMEM_SHARED((NP, D), jnp.float32),
            pltpu.SemaphoreType.DMA,
        ],
    )
    def prop(z_h, row_h, col_h, sp_h, rowv, colv, rows, acc, sem):
        c = lax.axis_index("c")
        s = lax.axis_index("s")
        w = _worker_id()
        pltpu.sync_copy(z_h.at[pl.ds(s * RPS, RPS)],
                        acc.at[pl.ds(s * RPS, RPS)])
        plsc.subcore_barrier()

        def body(bi, carry):
            base = w * EPT + bi * B
            pltpu.sync_copy(row_h.at[pl.ds(base, B)], rowv)
            pltpu.sync_copy(col_h.at[pl.ds(base, B)], colv)
            pltpu.async_copy(z_h.at[rowv], rows, sem).wait()
            pltpu.sync_copy(rows, acc.at[colv], add=True)
            return carry

        lax.fori_loop(0, NBE, body, 0)
        plsc.subcore_barrier()
        pltpu.sync_copy(acc.at[pl.ds(s * RPS, RPS)],
                        sp_h.at[c, pl.ds(s * RPS, RPS)])

    return prop


def _deg_of(degp_ref):
    return degp_ref[0, :, 0] + degp_ref[1, :, 0] + 1.0


def _scale_z0(degp_ref, x_ref, z_ref):
    r = lax.rsqrt(_deg_of(degp_ref))
    z_ref[...] = x_ref[...] * r[:, None]


def _combine_mid(degp_ref, sp_ref, z0_ref, z1_ref):
    dinv = 1.0 / _deg_of(degp_ref)
    z1_ref[...] = (sp_ref[0] + sp_ref[1] - z0_ref[...]) * dinv[:, None]


def _final(degp_ref, sp_ref, z1_ref, wt_ref, b_ref, out_ref):
    r = lax.rsqrt(_deg_of(degp_ref))
    x2 = (sp_ref[0] + sp_ref[1] - z1_ref[...]) * r[:, None]
    out_ref[...] = (
        jnp.dot(x2, wt_ref[...], preferred_element_type=jnp.float32)
        + b_ref[...]
    )


def kernel(x_indices, ei, emb_table, W, b):
    N, D = emb_table.shape
    OUT = W.shape[0]
    E = ei.shape[1]

    NP = (N // B + 1) * B
    assert NP % NW == 0 and NP % BLK == 0
    EPT = -(-E // (NW * B)) * B
    EP = NW * EPT

    row_pad = jnp.concatenate([ei[0], jnp.zeros((EP - E,), jnp.int32)])
    col_pad = jnp.concatenate([ei[1], jnp.full((EP - E,), N, jnp.int32)])
    xind_pad = jnp.concatenate(
        [x_indices.astype(jnp.int32), jnp.zeros((NP - N,), jnp.int32)])
    ones8 = jnp.ones((B, 8), jnp.float32)
    zer8 = jnp.zeros((NP // NS, 8), jnp.float32)
    wt = W.T
    b2 = b.reshape(1, OUT)

    x, degp = _make_pass_a(N, D, NP, EPT)(
        emb_table, xind_pad, col_pad, ones8, zer8)

    grid = (NP // BLK,)
    degp_spec = pl.BlockSpec((NC, BLK, 8), lambda i: (0, i, 0))
    row_spec = pl.BlockSpec((BLK, D), lambda i: (i, 0))
    sp_spec = pl.BlockSpec((NC, BLK, D), lambda i: (0, i, 0))

    z0 = pl.pallas_call(
        _scale_z0,
        grid=grid,
        in_specs=[degp_spec, row_spec],
        out_specs=row_spec,
        out_shape=jax.ShapeDtypeStruct((NP, D), jnp.float32),
    )(degp, x)

    prop = _make_prop(D, NP, EPT)
    sp1 = prop(z0, row_pad, col_pad)

    z1 = pl.pallas_call(
        _combine_mid,
        grid=grid,
        in_specs=[degp_spec, sp_spec, row_spec],
        out_specs=row_spec,
        out_shape=jax.ShapeDtypeStruct((NP, D), jnp.float32),
    )(degp, sp1, z0)

    sp2 = prop(z1, row_pad, col_pad)

    out = pl.pallas_call(
        _final,
        grid=grid,
        in_specs=[
            degp_spec, sp_spec, row_spec,
            pl.BlockSpec((D, OUT), lambda i: (0, 0)),
            pl.BlockSpec((1, OUT), lambda i: (0, 0)),
        ],
        out_specs=pl.BlockSpec((BLK, OUT), lambda i: (i, 0)),
        out_shape=jax.ShapeDtypeStruct((NP, OUT), jnp.float32),
    )(degp, sp2, z1, wt, b2)

    return out[:N]


# speedup vs baseline: 8.8452x; 8.8452x over previous
import functools

import jax
import jax.numpy as jnp
from jax import lax
from jax.experimental import pallas as pl
from jax.experimental.pallas import tpu as pltpu
from jax.experimental.pallas import tpu_sc as plsc

NC = 2
NS = 16
NW = NC * NS
B = 128
BLK = 128

_MESH = functools.partial(
    plsc.VectorSubcoreMesh,
    core_axis_name="c", subcore_axis_name="s", num_cores=NC, num_subcores=NS,
)


def _worker_id():
    return lax.axis_index("s") * NC + lax.axis_index("c")


WD = 128


def _make_pass_a(N, D, NP, EPT):
    RPS = NP // NS
    NBX = NP // B
    TX = (NBX + NW - 1) // NW
    NBE = EPT // B

    @functools.partial(
        pl.kernel,
        out_type=(
            jax.ShapeDtypeStruct((NP, D), jnp.float32),
            jax.ShapeDtypeStruct((NC, NP, WD), jnp.float32),
        ),
        mesh=_MESH(),
        scratch_types=[
            pltpu.VMEM((B,), jnp.int32),
            pltpu.VMEM((B, D), jnp.float32),
            pltpu.VMEM((B, WD), jnp.float32),
            pltpu.VMEM_SHARED((NP, WD), jnp.float32),
            pltpu.SemaphoreType.DMA,
        ],
    )
    def pass_a(emb_h, xind_h, col_h, ones_h, zer8_h, x_h, degp_h,
               idxv, rows, ones_v, deg_acc, sem):
        c = lax.axis_index("c")
        s = lax.axis_index("s")
        w = _worker_id()
        pltpu.sync_copy(zer8_h, deg_acc.at[pl.ds(s * RPS, RPS)])
        pltpu.sync_copy(ones_h, ones_v)
        plsc.subcore_barrier()

        def xgather(t, carry):
            bid = w + NW * t

            @pl.when(bid < NBX)
            def _():
                pltpu.sync_copy(xind_h.at[pl.ds(bid * B, B)], idxv)
                pltpu.async_copy(emb_h.at[idxv], rows, sem).wait()
                pltpu.sync_copy(rows, x_h.at[pl.ds(bid * B, B)])
            return carry

        lax.fori_loop(0, TX, xgather, 0)

        def deg_scatter(bi, carry):
            base = w * EPT + bi * B
            pltpu.sync_copy(col_h.at[pl.ds(base, B)], idxv)
            pltpu.sync_copy(ones_v, deg_acc.at[idxv], add=True)
            return carry

        lax.fori_loop(0, NBE, deg_scatter, 0)
        plsc.subcore_barrier()
        pltpu.sync_copy(deg_acc.at[pl.ds(s * RPS, RPS)],
                        degp_h.at[c, pl.ds(s * RPS, RPS)])

    return pass_a


def _make_prop(D, NP, EPT):
    RPS = NP // NS
    NBE = EPT // B

    @functools.partial(
        pl.kernel,
        out_type=jax.ShapeDtypeStruct((NC, NP, D), jnp.float32),
        mesh=_MESH(),
        scratch_types=[
            pltpu.VMEM((B,), jnp.int32),
            pltpu.VMEM((B,), jnp.int32),
            pltpu.VMEM((B, D), jnp.float32),
            pltpu.VMEM_SHARED((NP, D), jnp.float32),
            pltpu.SemaphoreType.DMA,
        ],
    )
    def prop(z_h, row_h, col_h, sp_h, rowv, colv, rows, acc, sem):
        c = lax.axis_index("c")
        s = lax.axis_index("s")
        w = _worker_id()
        pltpu.sync_copy(z_h.at[pl.ds(s * RPS, RPS)],
                        acc.at[pl.ds(s * RPS, RPS)])
        plsc.subcore_barrier()

        def body(bi, carry):
            base = w * EPT + bi * B
            pltpu.sync_copy(row_h.at[pl.ds(base, B)], rowv)
            pltpu.sync_copy(col_h.at[pl.ds(base, B)], colv)
            pltpu.async_copy(z_h.at[rowv], rows, sem).wait()
            pltpu.sync_copy(rows, acc.at[colv], add=True)
            return carry

        lax.fori_loop(0, NBE, body, 0)
        plsc.subcore_barrier()
        pltpu.sync_copy(acc.at[pl.ds(s * RPS, RPS)],
                        sp_h.at[c, pl.ds(s * RPS, RPS)])

    return prop


def _deg_of(degp_ref):
    return degp_ref[0, :, 0] + degp_ref[1, :, 0] + 1.0


def _scale_z0(degp_ref, x_ref, z_ref):
    r = lax.rsqrt(_deg_of(degp_ref))
    z_ref[...] = x_ref[...] * r[:, None]


def _combine_mid(degp_ref, sp_ref, z0_ref, z1_ref):
    dinv = 1.0 / _deg_of(degp_ref)
    z1_ref[...] = (sp_ref[0] + sp_ref[1] - z0_ref[...]) * dinv[:, None]


def _final(degp_ref, sp_ref, z1_ref, wt_ref, b_ref, out_ref):
    r = lax.rsqrt(_deg_of(degp_ref))
    x2 = (sp_ref[0] + sp_ref[1] - z1_ref[...]) * r[:, None]
    out_ref[...] = (
        jnp.dot(x2, wt_ref[...], preferred_element_type=jnp.float32)
        + b_ref[...]
    )


def kernel(x_indices, ei, emb_table, W, b):
    N, D = emb_table.shape
    OUT = W.shape[0]
    E = ei.shape[1]

    NP = (N // B + 1) * B
    assert NP % NW == 0 and NP % BLK == 0
    EPT = -(-E // (NW * B)) * B
    EP = NW * EPT

    row_pad = jnp.concatenate([ei[0], jnp.zeros((EP - E,), jnp.int32)])
    col_pad = jnp.concatenate([ei[1], jnp.full((EP - E,), N, jnp.int32)])
    xind_pad = jnp.concatenate(
        [x_indices.astype(jnp.int32), jnp.zeros((NP - N,), jnp.int32)])
    ones8 = jnp.ones((B, WD), jnp.float32)
    zer8 = jnp.zeros((NP // NS, WD), jnp.float32)
    wt = W.T
    b2 = b.reshape(1, OUT)

    x, degp = _make_pass_a(N, D, NP, EPT)(
        emb_table, xind_pad, col_pad, ones8, zer8)

    grid = (NP // BLK,)
    degp_spec = pl.BlockSpec((NC, BLK, WD), lambda i: (0, i, 0))
    row_spec = pl.BlockSpec((BLK, D), lambda i: (i, 0))
    sp_spec = pl.BlockSpec((NC, BLK, D), lambda i: (0, i, 0))

    z0 = pl.pallas_call(
        _scale_z0,
        grid=grid,
        in_specs=[degp_spec, row_spec],
        out_specs=row_spec,
        out_shape=jax.ShapeDtypeStruct((NP, D), jnp.float32),
    )(degp, x)

    prop = _make_prop(D, NP, EPT)
    sp1 = prop(z0, row_pad, col_pad)

    z1 = pl.pallas_call(
        _combine_mid,
        grid=grid,
        in_specs=[degp_spec, sp_spec, row_spec],
        out_specs=row_spec,
        out_shape=jax.ShapeDtypeStruct((NP, D), jnp.float32),
    )(degp, sp1, z0)

    sp2 = prop(z1, row_pad, col_pad)

    out = pl.pallas_call(
        _final,
        grid=grid,
        in_specs=[
            degp_spec, sp_spec, row_spec,
            pl.BlockSpec((D, OUT), lambda i: (0, 0)),
            pl.BlockSpec((1, OUT), lambda i: (0, 0)),
        ],
        out_specs=pl.BlockSpec((BLK, OUT), lambda i: (i, 0)),
        out_shape=jax.ShapeDtypeStruct((NP, OUT), jnp.float32),
    )(degp, sp2, z1, wt, b2)

    return out[:N]
